# Initial kernel scaffold; baseline (speedup 1.0000x reference)
import jax
import jax.numpy as jnp
from jax.experimental import pallas as pl


def kernel(inputs, Wg, We):
    raise NotImplementedError("write your pallas kernel here")


# speedup vs baseline: 1.0692x; 1.0692x over previous
import jax
import jax.numpy as jnp
from jax.experimental import pallas as pl
from jax.experimental.pallas import tpu as pltpu

DIM = 768
NUM_EXPERTS = 8
TOP_K = 2
NUM_TOKENS = 2048


def _gate_scales(x, wg, e):
    logits = jnp.dot(x, wg, preferred_element_type=jnp.float32)
    iota = jax.lax.broadcasted_iota(jnp.int32, logits.shape, 1)
    v1 = jnp.max(logits, axis=1, keepdims=True)
    i1 = jnp.min(jnp.where(logits == v1, iota, NUM_EXPERTS), axis=1, keepdims=True)
    l2 = jnp.where(iota == i1, -jnp.inf, logits)
    v2 = jnp.max(l2, axis=1, keepdims=True)
    i2 = jnp.min(jnp.where(l2 == v2, iota, NUM_EXPERTS), axis=1, keepdims=True)
    w1 = 1.0 / (1.0 + jnp.exp(v2 - v1))
    w2 = 1.0 - w1
    return jnp.where(i1 == e, w1, 0.0) + jnp.where(i2 == e, w2, 0.0)


def _moe_kernel(x_ref, wg_ref, we_ref, o_ref):
    e = pl.program_id(0)
    x = x_ref[...]
    scale = _gate_scales(x, wg_ref[...], e)

    @pl.when(e == 0)
    def _init():
        o_ref[...] = jnp.zeros_like(o_ref)

    o_ref[...] += scale * jnp.dot(x, we_ref[0], preferred_element_type=jnp.float32)


def kernel(inputs, Wg, We):
    return pl.pallas_call(
        _moe_kernel,
        grid=(NUM_EXPERTS,),
        in_specs=[
            pl.BlockSpec((NUM_TOKENS, DIM), lambda e: (0, 0)),
            pl.BlockSpec((DIM, NUM_EXPERTS), lambda e: (0, 0)),
            pl.BlockSpec((1, DIM, DIM), lambda e: (e, 0, 0)),
        ],
        out_specs=pl.BlockSpec((NUM_TOKENS, DIM), lambda e: (0, 0)),
        out_shape=jax.ShapeDtypeStruct((NUM_TOKENS, DIM), jnp.float32),
    )(inputs, Wg, We)


# speedup vs baseline: 1.0774x; 1.0076x over previous
import jax
import jax.numpy as jnp
from jax.experimental import pallas as pl
from jax.experimental.pallas import tpu as pltpu

DIM = 768
NUM_EXPERTS = 8
TOP_K = 2
NUM_TOKENS = 2048


def _gate_scales(x, wg, e):
    logits = jnp.dot(x, wg, preferred_element_type=jnp.float32)
    iota = jax.lax.broadcasted_iota(jnp.int32, logits.shape, 1)
    v1 = jnp.max(logits, axis=1, keepdims=True)
    i1 = jnp.min(jnp.where(logits == v1, iota, NUM_EXPERTS), axis=1, keepdims=True)
    l2 = jnp.where(iota == i1, -jnp.inf, logits)
    v2 = jnp.max(l2, axis=1, keepdims=True)
    i2 = jnp.min(jnp.where(l2 == v2, iota, NUM_EXPERTS), axis=1, keepdims=True)
    w1 = 1.0 / (1.0 + jnp.exp(v2 - v1))
    w2 = 1.0 - w1
    return jnp.where(i1 == e, w1, 0.0) + jnp.where(i2 == e, w2, 0.0)


def _moe_kernel(x_ref, wg_ref, we_ref, o_ref):
    e = pl.program_id(0)
    x = x_ref[...]
    scale = _gate_scales(x, wg_ref[...], e)

    @pl.when(e == 0)
    def _init():
        o_ref[...] = jnp.zeros_like(o_ref)

    xb = x.astype(jnp.bfloat16)
    wb = we_ref[0].astype(jnp.bfloat16)
    o_ref[...] += scale * jnp.dot(xb, wb, preferred_element_type=jnp.float32)


def kernel(inputs, Wg, We):
    return pl.pallas_call(
        _moe_kernel,
        grid=(NUM_EXPERTS,),
        in_specs=[
            pl.BlockSpec((NUM_TOKENS, DIM), lambda e: (0, 0)),
            pl.BlockSpec((DIM, NUM_EXPERTS), lambda e: (0, 0)),
            pl.BlockSpec((1, DIM, DIM), lambda e: (e, 0, 0)),
        ],
        out_specs=pl.BlockSpec((NUM_TOKENS, DIM), lambda e: (0, 0)),
        out_shape=jax.ShapeDtypeStruct((NUM_TOKENS, DIM), jnp.float32),
    )(inputs, Wg, We)


# speedup vs baseline: 1.5039x; 1.3958x over previous
import jax
import jax.numpy as jnp
from jax.experimental import pallas as pl
from jax.experimental.pallas import tpu as pltpu

DIM = 768
NUM_EXPERTS = 8
TOP_K = 2
NUM_TOKENS = 2048


def _moe_kernel(x_ref, wg_ref, we_ref, o_ref, i1_ref, i2_ref, w1_ref, w2_ref):
    e = pl.program_id(0)
    x = x_ref[...]

    @pl.when(e == 0)
    def _gate():
        logits = jnp.dot(x, wg_ref[...], preferred_element_type=jnp.float32)
        iota = jax.lax.broadcasted_iota(jnp.int32, logits.shape, 1)
        v1 = jnp.max(logits, axis=1, keepdims=True)
        i1 = jnp.min(jnp.where(logits == v1, iota, NUM_EXPERTS), axis=1, keepdims=True)
        l2 = jnp.where(iota == i1, -jnp.inf, logits)
        v2 = jnp.max(l2, axis=1, keepdims=True)
        i2 = jnp.min(jnp.where(l2 == v2, iota, NUM_EXPERTS), axis=1, keepdims=True)
        w1 = 1.0 / (1.0 + jnp.exp(v2 - v1))
        i1_ref[...] = i1
        i2_ref[...] = i2
        w1_ref[...] = w1
        w2_ref[...] = 1.0 - w1

    scale = jnp.where(i1_ref[...] == e, w1_ref[...], 0.0) + jnp.where(
        i2_ref[...] == e, w2_ref[...], 0.0
    )
    y = jnp.dot(
        x.astype(jnp.bfloat16),
        we_ref[0].astype(jnp.bfloat16),
        preferred_element_type=jnp.float32,
    )

    @pl.when(e == 0)
    def _first():
        o_ref[...] = scale * y

    @pl.when(e != 0)
    def _rest():
        o_ref[...] += scale * y


def kernel(inputs, Wg, We):
    return pl.pallas_call(
        _moe_kernel,
        grid=(NUM_EXPERTS,),
        in_specs=[
            pl.BlockSpec((NUM_TOKENS, DIM), lambda e: (0, 0)),
            pl.BlockSpec((DIM, NUM_EXPERTS), lambda e: (0, 0)),
            pl.BlockSpec((1, DIM, DIM), lambda e: (e, 0, 0)),
        ],
        out_specs=pl.BlockSpec((NUM_TOKENS, DIM), lambda e: (0, 0)),
        out_shape=jax.ShapeDtypeStruct((NUM_TOKENS, DIM), jnp.float32),
        scratch_shapes=[
            pltpu.VMEM((NUM_TOKENS, 1), jnp.int32),
            pltpu.VMEM((NUM_TOKENS, 1), jnp.int32),
            pltpu.VMEM((NUM_TOKENS, 1), jnp.float32),
            pltpu.VMEM((NUM_TOKENS, 1), jnp.float32),
        ],
    )(inputs, Wg, We)
